# Initial kernel scaffold; baseline (speedup 1.0000x reference)
import jax
import jax.numpy as jnp
from jax.experimental import pallas as pl


def kernel(x, edge_index, W1, b1, W2, b2, bern_w, Wf, bf, Wx, bx, lam):
    raise NotImplementedError("write your pallas kernel here")


# speedup vs baseline: 11.8430x; 11.8430x over previous
import functools

import jax
import jax.numpy as jnp
from jax import lax
from jax.experimental import pallas as pl
from jax.experimental.pallas import tpu as pltpu
from jax.experimental.pallas import tpu_sc as plsc

_NC = 2
_NS = 16
_NW = _NC * _NS
_CHUNK = 128
_QUAD = 4

_BERN_COEFFS = ((1.0, -2.0, 1.0), (0.0, 2.0, -2.0), (0.0, 0.0, 1.0))


def _sc_mesh():
    return plsc.VectorSubcoreMesh(core_axis_name="c", subcore_axis_name="s",
                                  num_cores=_NC, num_subcores=_NS)


@functools.lru_cache(maxsize=None)
def _deg_kernel(n_nodes, cpw, acc_rows):
    rpt = acc_rows // _NS
    nq = cpw // _QUAD

    def body(src3, dst3, zrows, out, sbuf, dbuf, ones_v, acc, sem):
        c = lax.axis_index("c")
        s = lax.axis_index("s")
        wid = s * _NC + c
        pltpu.sync_copy(src3.at[wid], sbuf)
        pltpu.sync_copy(dst3.at[wid], dbuf)

        def fill(i, _):
            ones_v[i, :] = jnp.full((16,), 1.0, jnp.float32)
            return 0
        lax.fori_loop(0, _CHUNK, fill, 0)

        def mark(j, _):
            for k in range(_CHUNK // 16):
                sv = sbuf[j, pl.ds(k * 16, 16)]
                dv = dbuf[j, pl.ds(k * 16, 16)]
                sbuf[j, pl.ds(k * 16, 16)] = jnp.where(sv == dv, n_nodes, sv)
            return 0
        lax.fori_loop(0, cpw, mark, 0)

        pltpu.sync_copy(zrows.at[pl.ds(s * rpt, rpt)],
                        acc.at[pl.ds(s * rpt, rpt)])
        plsc.subcore_barrier()

        def qloop(q, _):
            descs = [pltpu.async_copy(ones_v, acc.at[sbuf.at[q * _QUAD + t]],
                                      sem, add=True)
                     for t in range(_QUAD)]
            for d in descs:
                d.wait()
            return 0
        lax.fori_loop(0, nq, qloop, 0)
        plsc.subcore_barrier()
        pltpu.sync_copy(acc.at[pl.ds(s * rpt, rpt)],
                        out.at[c, pl.ds(s * rpt, rpt)])

    return pl.kernel(
        body,
        out_type=jax.ShapeDtypeStruct((_NC, acc_rows, 16), jnp.float32),
        mesh=_sc_mesh(),
        scratch_types=[
            pltpu.VMEM((cpw, _CHUNK), jnp.int32),
            pltpu.VMEM((cpw, _CHUNK), jnp.int32),
            pltpu.VMEM((_CHUNK, 16), jnp.float32),
            pltpu.VMEM_SHARED((acc_rows, 16), jnp.float32),
            pltpu.SemaphoreType.DMA,
        ],
        compiler_params=pltpu.CompilerParams(use_tc_tiling_on_sc=False),
    )


@functools.lru_cache(maxsize=None)
def _prop_kernel(n_nodes, cpw, acc_rows, hid):
    rpt = acc_rows // _NS
    nq = cpw // _QUAD

    def body(src3, dst3, y, zrows, out, sbuf, dbuf, rows, acc, sem):
        c = lax.axis_index("c")
        s = lax.axis_index("s")
        wid = s * _NC + c
        pltpu.sync_copy(src3.at[wid], sbuf)
        pltpu.sync_copy(dst3.at[wid], dbuf)

        def mark(j, _):
            for k in range(_CHUNK // 16):
                sv = sbuf[j, pl.ds(k * 16, 16)]
                dv = dbuf[j, pl.ds(k * 16, 16)]
                dbuf[j, pl.ds(k * 16, 16)] = jnp.where(sv == dv, n_nodes, dv)
            return 0
        lax.fori_loop(0, cpw, mark, 0)

        pltpu.sync_copy(zrows.at[pl.ds(s * rpt, rpt)],
                        acc.at[pl.ds(s * rpt, rpt)])
        plsc.subcore_barrier()

        def qloop(q, _):
            descs = [pltpu.async_copy(y.at[sbuf.at[q * _QUAD + t]],
                                      rows.at[t], sem)
                     for t in range(_QUAD)]
            for t in range(_QUAD):
                descs[t].wait()
                pltpu.sync_copy(rows.at[t], acc.at[dbuf.at[q * _QUAD + t]],
                                add=True)
            return 0
        lax.fori_loop(0, nq, qloop, 0)
        plsc.subcore_barrier()
        pltpu.sync_copy(acc.at[pl.ds(s * rpt, rpt)],
                        out.at[c, pl.ds(s * rpt, rpt)])

    return pl.kernel(
        body,
        out_type=jax.ShapeDtypeStruct((_NC, acc_rows, hid), jnp.float32),
        mesh=_sc_mesh(),
        scratch_types=[
            pltpu.VMEM((cpw, _CHUNK), jnp.int32),
            pltpu.VMEM((cpw, _CHUNK), jnp.int32),
            pltpu.VMEM((_QUAD, _CHUNK, hid), jnp.float32),
            pltpu.VMEM_SHARED((acc_rows, hid), jnp.float32),
            pltpu.SemaphoreType.DMA,
        ],
        compiler_params=pltpu.CompilerParams(use_tc_tiling_on_sc=False),
    )


def _xh_body(x_ref, w1_ref, b1_ref, w2_ref, b2_ref, o_ref):
    h = jnp.dot(x_ref[...], w1_ref[...], preferred_element_type=jnp.float32)
    h = jnp.maximum(h + b1_ref[...], 0.0)
    o_ref[...] = jnp.dot(h, w2_ref[...],
                         preferred_element_type=jnp.float32) + b2_ref[...]


def _scale_body(dega_ref, degb_ref, xh_ref, dis_ref, y0_ref):
    deg = dega_ref[...] + degb_ref[...]
    dis = jnp.where(deg > 0, 1.0 / jnp.sqrt(jnp.maximum(deg, 1e-12)), 0.0)
    dis_ref[...] = dis
    y0_ref[...] = xh_ref[...] * dis


def _combine_body(prev_ref, dis_ref, pa_ref, pb_ref, b_ref, y_ref):
    g = dis_ref[...] * (pa_ref[...] + pb_ref[...])
    b = 0.5 * prev_ref[...] - 0.5 * g
    b_ref[...] = b
    y_ref[...] = dis_ref[...] * b


def _combine_last_body(prev_ref, dis_ref, pa_ref, pb_ref, b_ref):
    g = dis_ref[...] * (pa_ref[...] + pb_ref[...])
    b_ref[...] = 0.5 * prev_ref[...] - 0.5 * g


def _fusion_body(xh_ref, b1_ref, b2_ref, wf_ref, bf_ref, wx_ref, bx_ref,
                 bw_ref, lam_ref, res_ref):
    xh = xh_ref[...]
    basis = (xh, b1_ref[...], b2_ref[...])
    wv = jax.nn.sigmoid(bw_ref[...])
    xp = jnp.tanh(jnp.dot(xh, wx_ref[...],
                          preferred_element_type=jnp.float32) + bx_ref[...])
    hs, ls = [], []
    for f in range(5):
        h = jnp.zeros_like(xh)
        for i in range(3):
            coef = jnp.zeros((1, 1), jnp.float32)
            for k in range(3):
                ck = _BERN_COEFFS[k][i]
                if ck != 0.0:
                    coef = coef + ck * wv[f:f + 1, k:k + 1]
            h = h + coef * basis[i]
        hs.append(h)
        hp = jnp.tanh(jnp.dot(h, wf_ref[...],
                              preferred_element_type=jnp.float32) + bf_ref[...])
        ls.append(jnp.sum(hp * xp, axis=1, keepdims=True))
    logits = jnp.concatenate(ls, axis=1)
    m = jnp.max(logits, axis=1, keepdims=True)
    e = jnp.exp(logits - m)
    score = e / jnp.sum(e, axis=1, keepdims=True)
    sidx = lax.broadcasted_iota(jnp.int32, (1, 5), 1)
    sf = jnp.where(sidx == 0, 1.0, jax.nn.sigmoid(lam_ref[...]))
    res = jnp.zeros_like(xh)
    for f in range(5):
        res = res + sf[:, f:f + 1] * score[:, f:f + 1] * hs[f]
    res_ref[...] = res


def _gram_body(a_ref, b_ref, o_ref):
    prod = lax.dot_general(a_ref[...], b_ref[...],
                           (((1,), (1,)), ((), ())),
                           preferred_element_type=jnp.float32)
    o_ref[...] = jnp.tanh(prod)


def _full_spec(shape):
    return pl.BlockSpec(shape, lambda *_: tuple(0 for _ in shape))


def _row_spec(blk, shape):
    nd = len(shape)
    return pl.BlockSpec((blk,) + shape[1:],
                        lambda i: (i,) + tuple(0 for _ in range(nd - 1)))


def kernel(x, edge_index, W1, b1, W2, b2, bern_w, Wf, bf, Wx, bx, lam):
    n, in_ch = x.shape
    hid = W2.shape[1]
    e = edge_index.shape[1]

    epw = ((e + _NW * _CHUNK - 1) // (_NW * _CHUNK)) * _CHUNK
    e_pad = _NW * epw
    cpw = epw // _CHUNK
    src = edge_index[0]
    dst = edge_index[1]
    if e_pad > e:
        zpad = jnp.zeros((e_pad - e,), jnp.int32)
        src = jnp.concatenate([src, zpad])
        dst = jnp.concatenate([dst, zpad])
    src3 = src.reshape(_NW, cpw, _CHUNK)
    dst3 = dst.reshape(_NW, cpw, _CHUNK)

    acc_rows = ((n + 1 + 127) // 128) * 128
    zeros16 = jnp.zeros((acc_rows, 16), jnp.float32)
    zeros_h = jnp.zeros((acc_rows, hid), jnp.float32)

    degp = _deg_kernel(n, cpw, acc_rows)(src3, dst3, zeros16)

    xh = pl.pallas_call(
        _xh_body,
        out_shape=jax.ShapeDtypeStruct((n, hid), jnp.float32),
        grid=(n // 2000,),
        in_specs=[_row_spec(2000, (n, in_ch)), _full_spec((in_ch, hid)),
                  _full_spec((1, hid)), _full_spec((hid, hid)),
                  _full_spec((1, hid))],
        out_specs=_row_spec(2000, (n, hid)),
    )(x, W1, b1.reshape(1, hid), W2, b2.reshape(1, hid))

    dega = degp[0, :n, 0:1]
    degb = degp[1, :n, 0:1]
    dis, y0 = pl.pallas_call(
        _scale_body,
        out_shape=[jax.ShapeDtypeStruct((n, 1), jnp.float32),
                   jax.ShapeDtypeStruct((n, hid), jnp.float32)],
        grid=(n // 2000,),
        in_specs=[_row_spec(2000, (n, 1)), _row_spec(2000, (n, 1)),
                  _row_spec(2000, (n, hid))],
        out_specs=[_row_spec(2000, (n, 1)), _row_spec(2000, (n, hid))],
    )(dega, degb, xh)

    s1 = _prop_kernel(n, cpw, acc_rows, hid)(src3, dst3, y0, zeros_h)
    b1_arr, y1 = pl.pallas_call(
        _combine_body,
        out_shape=[jax.ShapeDtypeStruct((n, hid), jnp.float32),
                   jax.ShapeDtypeStruct((n, hid), jnp.float32)],
        grid=(n // 2000,),
        in_specs=[_row_spec(2000, (n, hid)), _row_spec(2000, (n, 1)),
                  _row_spec(2000, (n, hid)), _row_spec(2000, (n, hid))],
        out_specs=[_row_spec(2000, (n, hid)), _row_spec(2000, (n, hid))],
    )(xh, dis, s1[0, :n], s1[1, :n])

    s2 = _prop_kernel(n, cpw, acc_rows, hid)(src3, dst3, y1, zeros_h)
    b2_arr = pl.pallas_call(
        _combine_last_body,
        out_shape=jax.ShapeDtypeStruct((n, hid), jnp.float32),
        grid=(n // 2000,),
        in_specs=[_row_spec(2000, (n, hid)), _row_spec(2000, (n, 1)),
                  _row_spec(2000, (n, hid)), _row_spec(2000, (n, hid))],
        out_specs=_row_spec(2000, (n, hid)),
    )(b1_arr, dis, s2[0, :n], s2[1, :n])

    res = pl.pallas_call(
        _fusion_body,
        out_shape=jax.ShapeDtypeStruct((n, hid), jnp.float32),
        grid=(n // 2000,),
        in_specs=[_row_spec(2000, (n, hid)), _row_spec(2000, (n, hid)),
                  _row_spec(2000, (n, hid)), _full_spec((hid, hid)),
                  _full_spec((1, hid)), _full_spec((hid, hid)),
                  _full_spec((1, hid)), _full_spec((5, 3)),
                  _full_spec((1, 5))],
        out_specs=_row_spec(2000, (n, hid)),
    )(xh, b1_arr, b2_arr, Wf, bf.reshape(1, hid), Wx, bx.reshape(1, hid),
      bern_w.reshape(5, 3), lam.reshape(1, 5))

    bm = 400
    out = pl.pallas_call(
        _gram_body,
        out_shape=jax.ShapeDtypeStruct((n, n), jnp.float32),
        grid=(n // bm,),
        in_specs=[pl.BlockSpec((bm, hid), lambda i: (i, 0)),
                  pl.BlockSpec((n, hid), lambda i: (0, 0))],
        out_specs=pl.BlockSpec((bm, n), lambda i: (i, 0)),
    )(res, res)
    return out


# speedup vs baseline: 12.2801x; 1.0369x over previous
import functools

import jax
import jax.numpy as jnp
from jax import lax
from jax.experimental import pallas as pl
from jax.experimental.pallas import tpu as pltpu
from jax.experimental.pallas import tpu_sc as plsc

_NC = 2
_NS = 16
_NW = _NC * _NS
_CHUNK = 128
_QUAD = 4

_BERN_COEFFS = ((1.0, -2.0, 1.0), (0.0, 2.0, -2.0), (0.0, 0.0, 1.0))


def _sc_mesh():
    return plsc.VectorSubcoreMesh(core_axis_name="c", subcore_axis_name="s",
                                  num_cores=_NC, num_subcores=_NS)


@functools.lru_cache(maxsize=None)
def _deg_kernel(n_nodes, cpw, acc_rows):
    rpt = acc_rows // _NS
    nq = cpw // _QUAD

    def body(src3, dst3, zrows, out, sbuf, dbuf, ones_v, acc, sem):
        c = lax.axis_index("c")
        s = lax.axis_index("s")
        wid = s * _NC + c
        pltpu.sync_copy(src3.at[wid], sbuf)
        pltpu.sync_copy(dst3.at[wid], dbuf)

        def fill(i, _):
            ones_v[i, :] = jnp.full((16,), 1.0, jnp.float32)
            return 0
        lax.fori_loop(0, _CHUNK, fill, 0)

        def mark(j, _):
            for k in range(_CHUNK // 16):
                sv = sbuf[j, pl.ds(k * 16, 16)]
                dv = dbuf[j, pl.ds(k * 16, 16)]
                sbuf[j, pl.ds(k * 16, 16)] = jnp.where(sv == dv, n_nodes, sv)
            return 0
        lax.fori_loop(0, cpw, mark, 0)

        pltpu.sync_copy(zrows.at[pl.ds(s * rpt, rpt)],
                        acc.at[pl.ds(s * rpt, rpt)])
        plsc.subcore_barrier()

        def qloop(q, _):
            descs = [pltpu.async_copy(ones_v, acc.at[sbuf.at[q * _QUAD + t]],
                                      sem, add=True)
                     for t in range(_QUAD)]
            for d in descs:
                d.wait()
            return 0
        lax.fori_loop(0, nq, qloop, 0)
        plsc.subcore_barrier()
        pltpu.sync_copy(acc.at[pl.ds(s * rpt, rpt)],
                        out.at[c, pl.ds(s * rpt, rpt)])

    return pl.kernel(
        body,
        out_type=jax.ShapeDtypeStruct((_NC, acc_rows, 16), jnp.float32),
        mesh=_sc_mesh(),
        scratch_types=[
            pltpu.VMEM((cpw, _CHUNK), jnp.int32),
            pltpu.VMEM((cpw, _CHUNK), jnp.int32),
            pltpu.VMEM((_CHUNK, 16), jnp.float32),
            pltpu.VMEM_SHARED((acc_rows, 16), jnp.float32),
            pltpu.SemaphoreType.DMA,
        ],
        compiler_params=pltpu.CompilerParams(use_tc_tiling_on_sc=False),
    )


@functools.lru_cache(maxsize=None)
def _prop_kernel(n_nodes, cpw, acc_rows, hid):
    rpt = acc_rows // _NS
    nq = cpw // _QUAD
    npair = nq // 2

    def body(src3, dst3, y, zrows, out, sbuf, dbuf, rows, acc,
             gsem_a, gsem_b, ssem):
        c = lax.axis_index("c")
        s = lax.axis_index("s")
        wid = s * _NC + c
        pltpu.sync_copy(src3.at[wid], sbuf)
        pltpu.sync_copy(dst3.at[wid], dbuf)

        def mark(j, _):
            for k in range(_CHUNK // 16):
                sv = sbuf[j, pl.ds(k * 16, 16)]
                dv = dbuf[j, pl.ds(k * 16, 16)]
                dbuf[j, pl.ds(k * 16, 16)] = jnp.where(sv == dv, n_nodes, dv)
            return 0
        lax.fori_loop(0, cpw, mark, 0)

        def gather(q, t):
            return pltpu.async_copy(y.at[sbuf.at[q * _QUAD + t]],
                                    rows.at[t], gsem_a)

        def gather_hi(q, t):
            return pltpu.async_copy(y.at[sbuf.at[q * _QUAD + t]],
                                    rows.at[_QUAD + t], gsem_b)

        def scatter(q, t, hi):
            slot = _QUAD + t if hi else t
            return pltpu.async_copy(rows.at[slot],
                                    acc.at[dbuf.at[q * _QUAD + t]],
                                    ssem, add=True)

        pltpu.sync_copy(zrows.at[pl.ds(s * rpt, rpt)],
                        acc.at[pl.ds(s * rpt, rpt)])
        for t in range(_QUAD):
            gather(0, t)
        plsc.subcore_barrier()

        def pair(p, _):
            qa = 2 * p
            qb = qa + 1
            gdesc_b = [gather_hi(qb, t) for t in range(_QUAD)]
            sdesc_a = []
            for t in range(_QUAD):
                pltpu.make_async_copy(y.at[sbuf.at[qa * _QUAD + t]],
                                      rows.at[t], gsem_a).wait()
                sdesc_a.append(scatter(qa, t, hi=False))
            for t in range(_QUAD):
                sdesc_a[t].wait()

            @pl.when(p + 1 < npair)
            def _():
                for t in range(_QUAD):
                    gather(2 * p + 2, t)
            sdesc_b = []
            for t in range(_QUAD):
                gdesc_b[t].wait()
                sdesc_b.append(scatter(qb, t, hi=True))
            for t in range(_QUAD):
                sdesc_b[t].wait()
            return 0
        lax.fori_loop(0, npair, pair, 0)
        plsc.subcore_barrier()
        pltpu.sync_copy(acc.at[pl.ds(s * rpt, rpt)],
                        out.at[c, pl.ds(s * rpt, rpt)])

    return pl.kernel(
        body,
        out_type=jax.ShapeDtypeStruct((_NC, acc_rows, hid), jnp.float32),
        mesh=_sc_mesh(),
        scratch_types=[
            pltpu.VMEM((cpw, _CHUNK), jnp.int32),
            pltpu.VMEM((cpw, _CHUNK), jnp.int32),
            pltpu.VMEM((2 * _QUAD, _CHUNK, hid), jnp.float32),
            pltpu.VMEM_SHARED((acc_rows, hid), jnp.float32),
            pltpu.SemaphoreType.DMA,
            pltpu.SemaphoreType.DMA,
            pltpu.SemaphoreType.DMA,
        ],
        compiler_params=pltpu.CompilerParams(use_tc_tiling_on_sc=False),
    )


def _xh_body(x_ref, w1_ref, b1_ref, w2_ref, b2_ref, o_ref):
    h = jnp.dot(x_ref[...], w1_ref[...], preferred_element_type=jnp.float32)
    h = jnp.maximum(h + b1_ref[...], 0.0)
    o_ref[...] = jnp.dot(h, w2_ref[...],
                         preferred_element_type=jnp.float32) + b2_ref[...]


def _scale_body(dega_ref, degb_ref, xh_ref, dis_ref, y0_ref):
    deg = dega_ref[...] + degb_ref[...]
    dis = jnp.where(deg > 0, 1.0 / jnp.sqrt(jnp.maximum(deg, 1e-12)), 0.0)
    dis_ref[...] = dis
    y0_ref[...] = xh_ref[...] * dis


def _combine_body(prev_ref, dis_ref, pa_ref, pb_ref, b_ref, y_ref):
    g = dis_ref[...] * (pa_ref[...] + pb_ref[...])
    b = 0.5 * prev_ref[...] - 0.5 * g
    b_ref[...] = b
    y_ref[...] = dis_ref[...] * b


def _fusion_body(xh_ref, b1_ref, dis_ref, pa_ref, pb_ref,
                 wf_ref, bf_ref, wx_ref, bx_ref,
                 bw_ref, lam_ref, res_ref):
    xh = xh_ref[...]
    b1v = b1_ref[...]
    b2v = 0.5 * b1v - 0.5 * dis_ref[...] * (pa_ref[...] + pb_ref[...])
    basis = (xh, b1v, b2v)
    wv = jax.nn.sigmoid(bw_ref[...])
    xp = jnp.tanh(jnp.dot(xh, wx_ref[...],
                          preferred_element_type=jnp.float32) + bx_ref[...])
    hs, ls = [], []
    for f in range(5):
        h = jnp.zeros_like(xh)
        for i in range(3):
            coef = jnp.zeros((1, 1), jnp.float32)
            for k in range(3):
                ck = _BERN_COEFFS[k][i]
                if ck != 0.0:
                    coef = coef + ck * wv[f:f + 1, k:k + 1]
            h = h + coef * basis[i]
        hs.append(h)
        hp = jnp.tanh(jnp.dot(h, wf_ref[...],
                              preferred_element_type=jnp.float32) + bf_ref[...])
        ls.append(jnp.sum(hp * xp, axis=1, keepdims=True))
    logits = jnp.concatenate(ls, axis=1)
    m = jnp.max(logits, axis=1, keepdims=True)
    e = jnp.exp(logits - m)
    score = e / jnp.sum(e, axis=1, keepdims=True)
    sidx = lax.broadcasted_iota(jnp.int32, (1, 5), 1)
    sf = jnp.where(sidx == 0, 1.0, jax.nn.sigmoid(lam_ref[...]))
    res = jnp.zeros_like(xh)
    for f in range(5):
        res = res + sf[:, f:f + 1] * score[:, f:f + 1] * hs[f]
    res_ref[...] = res


def _gram_body(a_ref, b_ref, o_ref):
    prod = lax.dot_general(a_ref[...], b_ref[...],
                           (((1,), (1,)), ((), ())),
                           preferred_element_type=jnp.float32)
    o_ref[...] = jnp.tanh(prod)


def _full_spec(shape):
    return pl.BlockSpec(shape, lambda *_: tuple(0 for _ in shape))


def _row_spec(blk, shape):
    nd = len(shape)
    return pl.BlockSpec((blk,) + shape[1:],
                        lambda i: (i,) + tuple(0 for _ in range(nd - 1)))


def kernel(x, edge_index, W1, b1, W2, b2, bern_w, Wf, bf, Wx, bx, lam):
    n, in_ch = x.shape
    hid = W2.shape[1]
    e = edge_index.shape[1]

    epw = ((e + _NW * _CHUNK - 1) // (_NW * _CHUNK)) * _CHUNK
    e_pad = _NW * epw
    cpw = epw // _CHUNK
    src = edge_index[0]
    dst = edge_index[1]
    if e_pad > e:
        zpad = jnp.zeros((e_pad - e,), jnp.int32)
        src = jnp.concatenate([src, zpad])
        dst = jnp.concatenate([dst, zpad])
    src3 = src.reshape(_NW, cpw, _CHUNK)
    dst3 = dst.reshape(_NW, cpw, _CHUNK)

    acc_rows = ((n + 1 + 127) // 128) * 128
    zeros16 = jnp.zeros((acc_rows, 16), jnp.float32)
    zeros_h = jnp.zeros((acc_rows, hid), jnp.float32)

    degp = _deg_kernel(n, cpw, acc_rows)(src3, dst3, zeros16)

    xh = pl.pallas_call(
        _xh_body,
        out_shape=jax.ShapeDtypeStruct((n, hid), jnp.float32),
        grid=(n // 2000,),
        in_specs=[_row_spec(2000, (n, in_ch)), _full_spec((in_ch, hid)),
                  _full_spec((1, hid)), _full_spec((hid, hid)),
                  _full_spec((1, hid))],
        out_specs=_row_spec(2000, (n, hid)),
    )(x, W1, b1.reshape(1, hid), W2, b2.reshape(1, hid))

    dega = degp[0, :n, 0:1]
    degb = degp[1, :n, 0:1]
    dis, y0 = pl.pallas_call(
        _scale_body,
        out_shape=[jax.ShapeDtypeStruct((n, 1), jnp.float32),
                   jax.ShapeDtypeStruct((n, hid), jnp.float32)],
        grid=(n // 2000,),
        in_specs=[_row_spec(2000, (n, 1)), _row_spec(2000, (n, 1)),
                  _row_spec(2000, (n, hid))],
        out_specs=[_row_spec(2000, (n, 1)), _row_spec(2000, (n, hid))],
    )(dega, degb, xh)

    s1 = _prop_kernel(n, cpw, acc_rows, hid)(src3, dst3, y0, zeros_h)
    b1_arr, y1 = pl.pallas_call(
        _combine_body,
        out_shape=[jax.ShapeDtypeStruct((n, hid), jnp.float32),
                   jax.ShapeDtypeStruct((n, hid), jnp.float32)],
        grid=(n // 2000,),
        in_specs=[_row_spec(2000, (n, hid)), _row_spec(2000, (n, 1)),
                  _row_spec(2000, (n, hid)), _row_spec(2000, (n, hid))],
        out_specs=[_row_spec(2000, (n, hid)), _row_spec(2000, (n, hid))],
    )(xh, dis, s1[0, :n], s1[1, :n])

    s2 = _prop_kernel(n, cpw, acc_rows, hid)(src3, dst3, y1, zeros_h)
    res = pl.pallas_call(
        _fusion_body,
        out_shape=jax.ShapeDtypeStruct((n, hid), jnp.float32),
        grid=(n // 2000,),
        in_specs=[_row_spec(2000, (n, hid)), _row_spec(2000, (n, hid)),
                  _row_spec(2000, (n, 1)), _row_spec(2000, (n, hid)),
                  _row_spec(2000, (n, hid)), _full_spec((hid, hid)),
                  _full_spec((1, hid)), _full_spec((hid, hid)),
                  _full_spec((1, hid)), _full_spec((5, 3)),
                  _full_spec((1, 5))],
        out_specs=_row_spec(2000, (n, hid)),
    )(xh, b1_arr, dis, s2[0, :n], s2[1, :n], Wf, bf.reshape(1, hid),
      Wx, bx.reshape(1, hid), bern_w.reshape(5, 3), lam.reshape(1, 5))

    bm = 400
    out = pl.pallas_call(
        _gram_body,
        out_shape=jax.ShapeDtypeStruct((n, n), jnp.float32),
        grid=(n // bm,),
        in_specs=[pl.BlockSpec((bm, hid), lambda i: (i, 0)),
                  pl.BlockSpec((n, hid), lambda i: (0, 0))],
        out_specs=pl.BlockSpec((bm, n), lambda i: (i, 0)),
    )(res, res)
    return out


# speedup vs baseline: 17.2198x; 1.4023x over previous
import functools

import jax
import jax.numpy as jnp
from jax import lax
from jax.experimental import pallas as pl
from jax.experimental.pallas import tpu as pltpu
from jax.experimental.pallas import tpu_sc as plsc

_NC = 2
_NS = 16
_NW = _NC * _NS
_CHUNK = 128
_QUAD = 4

_BERN_COEFFS = ((1.0, -2.0, 1.0), (0.0, 2.0, -2.0), (0.0, 0.0, 1.0))


def _sc_mesh():
    return plsc.VectorSubcoreMesh(core_axis_name="c", subcore_axis_name="s",
                                  num_cores=_NC, num_subcores=_NS)


@functools.lru_cache(maxsize=None)
def _deg_kernel(n_nodes, cpw, acc_rows):
    rpt = acc_rows // _NS
    nq = cpw // _QUAD

    def body(src3, dst3, zrows, out, sbuf, dbuf, ones_v, acc, sem):
        c = lax.axis_index("c")
        s = lax.axis_index("s")
        wid = s * _NC + c
        pltpu.sync_copy(src3.at[wid], sbuf)
        pltpu.sync_copy(dst3.at[wid], dbuf)

        def fill(i, _):
            ones_v[i, :] = jnp.full((16,), 1.0, jnp.float32)
            return 0
        lax.fori_loop(0, _CHUNK, fill, 0)

        def mark(j, _):
            for k in range(_CHUNK // 16):
                sv = sbuf[j, pl.ds(k * 16, 16)]
                dv = dbuf[j, pl.ds(k * 16, 16)]
                sbuf[j, pl.ds(k * 16, 16)] = jnp.where(sv == dv, n_nodes, sv)
            return 0
        lax.fori_loop(0, cpw, mark, 0)

        pltpu.sync_copy(zrows.at[pl.ds(s * rpt, rpt)],
                        acc.at[pl.ds(s * rpt, rpt)])
        plsc.subcore_barrier()

        def qloop(q, _):
            descs = [pltpu.async_copy(ones_v, acc.at[sbuf.at[q * _QUAD + t]],
                                      sem, add=True)
                     for t in range(_QUAD)]
            for d in descs:
                d.wait()
            return 0
        lax.fori_loop(0, nq, qloop, 0)
        plsc.subcore_barrier()
        pltpu.sync_copy(acc.at[pl.ds(s * rpt, rpt)],
                        out.at[c, pl.ds(s * rpt, rpt)])

    return pl.kernel(
        body,
        out_type=jax.ShapeDtypeStruct((_NC, acc_rows, 16), jnp.float32),
        mesh=_sc_mesh(),
        scratch_types=[
            pltpu.VMEM((cpw, _CHUNK), jnp.int32),
            pltpu.VMEM((cpw, _CHUNK), jnp.int32),
            pltpu.VMEM((_CHUNK, 16), jnp.float32),
            pltpu.VMEM_SHARED((acc_rows, 16), jnp.float32),
            pltpu.SemaphoreType.DMA,
        ],
        compiler_params=pltpu.CompilerParams(use_tc_tiling_on_sc=False),
    )


@functools.lru_cache(maxsize=None)
def _prop_kernel(n_nodes, cpw, acc_rows, hid):
    hh = hid // 2
    rpt = acc_rows // _NS
    ypt = n_nodes // _NS
    nq = cpw // _QUAD
    npair = nq // 2

    zrep = 4
    zrows_n = rpt // zrep

    def body(src3, dst3, y_hbm, out, sbuf, dbuf, rows, zbuf, ytab, acc,
             gsem_a, gsem_b, ssem):
        c = lax.axis_index("c")
        s = lax.axis_index("s")
        pltpu.sync_copy(y_hbm.at[c, pl.ds(s * ypt, ypt)],
                        ytab.at[pl.ds(s * ypt, ypt)])
        pltpu.sync_copy(src3.at[s], sbuf)
        pltpu.sync_copy(dst3.at[s], dbuf)

        def mark(j, _):
            for k in range(_CHUNK // 16):
                sv = sbuf[j, pl.ds(k * 16, 16)]
                dv = dbuf[j, pl.ds(k * 16, 16)]
                dbuf[j, pl.ds(k * 16, 16)] = jnp.where(sv == dv, n_nodes, dv)
            return 0
        lax.fori_loop(0, cpw, mark, 0)

        def gather(q, t):
            return pltpu.async_copy(ytab.at[sbuf.at[q * _QUAD + t]],
                                    rows.at[t], gsem_a)

        def gather_hi(q, t):
            return pltpu.async_copy(ytab.at[sbuf.at[q * _QUAD + t]],
                                    rows.at[_QUAD + t], gsem_b)

        def scatter(q, t, hi):
            slot = _QUAD + t if hi else t
            return pltpu.async_copy(rows.at[slot],
                                    acc.at[dbuf.at[q * _QUAD + t]],
                                    ssem, add=True)

        def zfill(i, _):
            for k in range(hh // 16):
                zbuf[i, pl.ds(k * 16, 16)] = jnp.zeros((16,), jnp.float32)
            return 0
        lax.fori_loop(0, zrows_n, zfill, 0)
        for r in range(zrep):
            pltpu.sync_copy(zbuf, acc.at[pl.ds(s * rpt + r * zrows_n,
                                               zrows_n)])
        plsc.subcore_barrier()
        for t in range(_QUAD):
            gather(0, t)

        def pair(p, _):
            qa = 2 * p
            qb = qa + 1
            gdesc_b = [gather_hi(qb, t) for t in range(_QUAD)]
            sdesc_a = []
            for t in range(_QUAD):
                pltpu.make_async_copy(ytab.at[sbuf.at[qa * _QUAD + t]],
                                      rows.at[t], gsem_a).wait()
                sdesc_a.append(scatter(qa, t, hi=False))
            for t in range(_QUAD):
                sdesc_a[t].wait()

            @pl.when(p + 1 < npair)
            def _():
                for t in range(_QUAD):
                    gather(2 * p + 2, t)
            sdesc_b = []
            for t in range(_QUAD):
                gdesc_b[t].wait()
                sdesc_b.append(scatter(qb, t, hi=True))
            for t in range(_QUAD):
                sdesc_b[t].wait()
            return 0
        lax.fori_loop(0, npair, pair, 0)
        plsc.subcore_barrier()
        pltpu.sync_copy(acc.at[pl.ds(s * rpt, rpt)],
                        out.at[c, pl.ds(s * rpt, rpt)])

    return pl.kernel(
        body,
        out_type=jax.ShapeDtypeStruct((_NC, acc_rows, hh), jnp.float32),
        mesh=_sc_mesh(),
        scratch_types=[
            pltpu.VMEM((cpw, _CHUNK), jnp.int32),
            pltpu.VMEM((cpw, _CHUNK), jnp.int32),
            pltpu.VMEM((2 * _QUAD, _CHUNK, hh), jnp.float32),
            pltpu.VMEM((rpt // 4, hh), jnp.float32),
            pltpu.VMEM_SHARED((n_nodes, hh), jnp.float32),
            pltpu.VMEM_SHARED((acc_rows, hh), jnp.float32),
            pltpu.SemaphoreType.DMA,
            pltpu.SemaphoreType.DMA,
            pltpu.SemaphoreType.DMA,
        ],
        compiler_params=pltpu.CompilerParams(use_tc_tiling_on_sc=False),
    )


def _xh_body(x_ref, w1_ref, b1_ref, w2_ref, b2_ref, o_ref):
    h = jnp.dot(x_ref[...], w1_ref[...], preferred_element_type=jnp.float32)
    h = jnp.maximum(h + b1_ref[...], 0.0)
    o_ref[...] = jnp.dot(h, w2_ref[...],
                         preferred_element_type=jnp.float32) + b2_ref[...]


def _scale_body(dega_ref, degb_ref, xh_ref, dis_ref, ya_ref, yb_ref):
    deg = dega_ref[...] + degb_ref[...]
    dis = jnp.where(deg > 0, 1.0 / jnp.sqrt(jnp.maximum(deg, 1e-12)), 0.0)
    dis_ref[...] = dis
    y = xh_ref[...] * dis
    hh = y.shape[1] // 2
    ya_ref[...] = y[:, :hh]
    yb_ref[...] = y[:, hh:]


def _combine_body(prev_ref, dis_ref, pa_ref, pb_ref, b_ref, ya_ref, yb_ref):
    g = dis_ref[...] * jnp.concatenate([pa_ref[...], pb_ref[...]], axis=1)
    b = 0.5 * prev_ref[...] - 0.5 * g
    b_ref[...] = b
    y = dis_ref[...] * b
    hh = y.shape[1] // 2
    ya_ref[...] = y[:, :hh]
    yb_ref[...] = y[:, hh:]


def _fusion_body(xh_ref, b1_ref, dis_ref, pa_ref, pb_ref,
                 wf_ref, bf_ref, wx_ref, bx_ref,
                 bw_ref, lam_ref, res_ref):
    xh = xh_ref[...]
    b1v = b1_ref[...]
    b2v = 0.5 * b1v - 0.5 * dis_ref[...] * jnp.concatenate(
        [pa_ref[...], pb_ref[...]], axis=1)
    basis = (xh, b1v, b2v)
    wv = jax.nn.sigmoid(bw_ref[...])
    xp = jnp.tanh(jnp.dot(xh, wx_ref[...],
                          preferred_element_type=jnp.float32) + bx_ref[...])
    hs, ls = [], []
    for f in range(5):
        h = jnp.zeros_like(xh)
        for i in range(3):
            coef = jnp.zeros((1, 1), jnp.float32)
            for k in range(3):
                ck = _BERN_COEFFS[k][i]
                if ck != 0.0:
                    coef = coef + ck * wv[f:f + 1, k:k + 1]
            h = h + coef * basis[i]
        hs.append(h)
        hp = jnp.tanh(jnp.dot(h, wf_ref[...],
                              preferred_element_type=jnp.float32) + bf_ref[...])
        ls.append(jnp.sum(hp * xp, axis=1, keepdims=True))
    logits = jnp.concatenate(ls, axis=1)
    m = jnp.max(logits, axis=1, keepdims=True)
    e = jnp.exp(logits - m)
    score = e / jnp.sum(e, axis=1, keepdims=True)
    sidx = lax.broadcasted_iota(jnp.int32, (1, 5), 1)
    sf = jnp.where(sidx == 0, 1.0, jax.nn.sigmoid(lam_ref[...]))
    res = jnp.zeros_like(xh)
    for f in range(5):
        res = res + sf[:, f:f + 1] * score[:, f:f + 1] * hs[f]
    res_ref[...] = res


def _gram_body(a_ref, b_ref, o_ref):
    prod = lax.dot_general(a_ref[...], b_ref[...],
                           (((1,), (1,)), ((), ())),
                           preferred_element_type=jnp.float32)
    o_ref[...] = jnp.tanh(prod)


def _full_spec(shape):
    return pl.BlockSpec(shape, lambda *_: tuple(0 for _ in shape))


def _row_spec(blk, shape):
    nd = len(shape)
    return pl.BlockSpec((blk,) + shape[1:],
                        lambda i: (i,) + tuple(0 for _ in range(nd - 1)))


def kernel(x, edge_index, W1, b1, W2, b2, bern_w, Wf, bf, Wx, bx, lam):
    n, in_ch = x.shape
    hid = W2.shape[1]
    e = edge_index.shape[1]

    epw = ((e + _NW * _CHUNK - 1) // (_NW * _CHUNK)) * _CHUNK
    e_pad = _NW * epw
    cpw = epw // _CHUNK
    src = edge_index[0]
    dst = edge_index[1]
    if e_pad > e:
        zpad = jnp.zeros((e_pad - e,), jnp.int32)
        src = jnp.concatenate([src, zpad])
        dst = jnp.concatenate([dst, zpad])
    src3 = src.reshape(_NW, cpw, _CHUNK)
    dst3 = dst.reshape(_NW, cpw, _CHUNK)
    cpw2 = e_pad // (_NS * _CHUNK)
    src3p = src.reshape(_NS, cpw2, _CHUNK)
    dst3p = dst.reshape(_NS, cpw2, _CHUNK)
    hh = hid // 2

    acc_rows = ((n + 1 + 127) // 128) * 128
    zeros16 = jnp.zeros((acc_rows, 16), jnp.float32)

    degp = _deg_kernel(n, cpw, acc_rows)(src3, dst3, zeros16)

    xh = pl.pallas_call(
        _xh_body,
        out_shape=jax.ShapeDtypeStruct((n, hid), jnp.float32),
        grid=(n // 2000,),
        in_specs=[_row_spec(2000, (n, in_ch)), _full_spec((in_ch, hid)),
                  _full_spec((1, hid)), _full_spec((hid, hid)),
                  _full_spec((1, hid))],
        out_specs=_row_spec(2000, (n, hid)),
    )(x, W1, b1.reshape(1, hid), W2, b2.reshape(1, hid))

    dega = degp[0, :n, 0:1]
    degb = degp[1, :n, 0:1]
    dis, y0a, y0b = pl.pallas_call(
        _scale_body,
        out_shape=[jax.ShapeDtypeStruct((n, 1), jnp.float32),
                   jax.ShapeDtypeStruct((n, hh), jnp.float32),
                   jax.ShapeDtypeStruct((n, hh), jnp.float32)],
        grid=(n // 2000,),
        in_specs=[_row_spec(2000, (n, 1)), _row_spec(2000, (n, 1)),
                  _row_spec(2000, (n, hid))],
        out_specs=[_row_spec(2000, (n, 1)), _row_spec(2000, (n, hh)),
                   _row_spec(2000, (n, hh))],
    )(dega, degb, xh)

    s1 = _prop_kernel(n, cpw2, acc_rows, hid)(src3p, dst3p,
                                              jnp.stack([y0a, y0b]))
    b1_arr, y1a, y1b = pl.pallas_call(
        _combine_body,
        out_shape=[jax.ShapeDtypeStruct((n, hid), jnp.float32),
                   jax.ShapeDtypeStruct((n, hh), jnp.float32),
                   jax.ShapeDtypeStruct((n, hh), jnp.float32)],
        grid=(n // 2000,),
        in_specs=[_row_spec(2000, (n, hid)), _row_spec(2000, (n, 1)),
                  _row_spec(2000, (n, hh)), _row_spec(2000, (n, hh))],
        out_specs=[_row_spec(2000, (n, hid)), _row_spec(2000, (n, hh)),
                   _row_spec(2000, (n, hh))],
    )(xh, dis, s1[0, :n], s1[1, :n])

    s2 = _prop_kernel(n, cpw2, acc_rows, hid)(src3p, dst3p,
                                              jnp.stack([y1a, y1b]))
    res = pl.pallas_call(
        _fusion_body,
        out_shape=jax.ShapeDtypeStruct((n, hid), jnp.float32),
        grid=(n // 2000,),
        in_specs=[_row_spec(2000, (n, hid)), _row_spec(2000, (n, hid)),
                  _row_spec(2000, (n, 1)), _row_spec(2000, (n, hh)),
                  _row_spec(2000, (n, hh)), _full_spec((hid, hid)),
                  _full_spec((1, hid)), _full_spec((hid, hid)),
                  _full_spec((1, hid)), _full_spec((5, 3)),
                  _full_spec((1, 5))],
        out_specs=_row_spec(2000, (n, hid)),
    )(xh, b1_arr, dis, s2[0, :n], s2[1, :n], Wf, bf.reshape(1, hid),
      Wx, bx.reshape(1, hid), bern_w.reshape(5, 3), lam.reshape(1, 5))

    bm = 400
    out = pl.pallas_call(
        _gram_body,
        out_shape=jax.ShapeDtypeStruct((n, n), jnp.float32),
        grid=(n // bm,),
        in_specs=[pl.BlockSpec((bm, hid), lambda i: (i, 0)),
                  pl.BlockSpec((n, hid), lambda i: (0, 0))],
        out_specs=pl.BlockSpec((bm, n), lambda i: (i, 0)),
    )(res, res)
    return out


# speedup vs baseline: 19.0401x; 1.1057x over previous
import functools

import jax
import jax.numpy as jnp
from jax import lax
from jax.experimental import pallas as pl
from jax.experimental.pallas import tpu as pltpu
from jax.experimental.pallas import tpu_sc as plsc

_NC = 2
_NS = 16
_NW = _NC * _NS
_CHUNK = 128
_QUAD = 4

_BERN_COEFFS = ((1.0, -2.0, 1.0), (0.0, 2.0, -2.0), (0.0, 0.0, 1.0))


def _sc_mesh():
    return plsc.VectorSubcoreMesh(core_axis_name="c", subcore_axis_name="s",
                                  num_cores=_NC, num_subcores=_NS)


@functools.lru_cache(maxsize=None)
def _deg_kernel(n_nodes, cpw, acc_rows):
    rpt = acc_rows // _NS
    nq = cpw // _QUAD

    def body(src3, dst3, zrows, out, sbuf, dbuf, ones_v, acc, sem):
        c = lax.axis_index("c")
        s = lax.axis_index("s")
        wid = s * _NC + c
        pltpu.sync_copy(src3.at[wid], sbuf)
        pltpu.sync_copy(dst3.at[wid], dbuf)

        def fill(i, _):
            ones_v[i, :] = jnp.full((16,), 1.0, jnp.float32)
            return 0
        lax.fori_loop(0, _CHUNK, fill, 0)

        def mark(j, _):
            for k in range(_CHUNK // 16):
                sv = sbuf[j, pl.ds(k * 16, 16)]
                dv = dbuf[j, pl.ds(k * 16, 16)]
                sbuf[j, pl.ds(k * 16, 16)] = jnp.where(sv == dv, n_nodes, sv)
            return 0
        lax.fori_loop(0, cpw, mark, 0)

        pltpu.sync_copy(zrows.at[pl.ds(s * rpt, rpt)],
                        acc.at[pl.ds(s * rpt, rpt)])
        plsc.subcore_barrier()

        def qloop(q, _):
            descs = [pltpu.async_copy(ones_v, acc.at[sbuf.at[q * _QUAD + t]],
                                      sem, add=True)
                     for t in range(_QUAD)]
            for d in descs:
                d.wait()
            return 0
        lax.fori_loop(0, nq, qloop, 0)
        plsc.subcore_barrier()
        pltpu.sync_copy(acc.at[pl.ds(s * rpt, rpt)],
                        out.at[c, pl.ds(s * rpt, rpt)])

    return pl.kernel(
        body,
        out_type=jax.ShapeDtypeStruct((_NC, acc_rows, 16), jnp.float32),
        mesh=_sc_mesh(),
        scratch_types=[
            pltpu.VMEM((cpw, _CHUNK), jnp.int32),
            pltpu.VMEM((cpw, _CHUNK), jnp.int32),
            pltpu.VMEM((_CHUNK, 16), jnp.float32),
            pltpu.VMEM_SHARED((acc_rows, 16), jnp.float32),
            pltpu.SemaphoreType.DMA,
        ],
        compiler_params=pltpu.CompilerParams(use_tc_tiling_on_sc=False),
    )


@functools.lru_cache(maxsize=None)
def _prop_kernel(n_nodes, cpw, acc_rows, hid):
    hh = hid // 2
    rpt = acc_rows // _NS
    ypt = n_nodes // _NS
    nq = cpw // _QUAD
    npair = nq // 2

    zrep = 4
    zrows_n = rpt // zrep

    def body(src3, dst3, y_hbm, out, sbuf, dbuf, rows, zbuf, ytab, acc,
             gsem_a, gsem_b, ssem):
        c = lax.axis_index("c")
        s = lax.axis_index("s")
        pltpu.sync_copy(y_hbm.at[c, pl.ds(s * ypt, ypt)],
                        ytab.at[pl.ds(s * ypt, ypt)])
        pltpu.sync_copy(src3.at[s], sbuf)
        pltpu.sync_copy(dst3.at[s], dbuf)

        def mark(j, _):
            for k in range(_CHUNK // 16):
                sv = sbuf[j, pl.ds(k * 16, 16)]
                dv = dbuf[j, pl.ds(k * 16, 16)]
                dbuf[j, pl.ds(k * 16, 16)] = jnp.where(sv == dv, n_nodes, dv)
            return 0
        lax.fori_loop(0, cpw, mark, 0)

        def gather(q, t):
            return pltpu.async_copy(ytab.at[sbuf.at[q * _QUAD + t]],
                                    rows.at[t], gsem_a)

        def gather_hi(q, t):
            return pltpu.async_copy(ytab.at[sbuf.at[q * _QUAD + t]],
                                    rows.at[_QUAD + t], gsem_b)

        def scatter(q, t, hi):
            slot = _QUAD + t if hi else t
            return pltpu.async_copy(rows.at[slot],
                                    acc.at[dbuf.at[q * _QUAD + t]],
                                    ssem, add=True)

        def zfill(i, _):
            for k in range(hh // 16):
                zbuf[i, pl.ds(k * 16, 16)] = jnp.zeros((16,), jnp.float32)
            return 0
        lax.fori_loop(0, zrows_n, zfill, 0)
        for r in range(zrep):
            pltpu.sync_copy(zbuf, acc.at[pl.ds(s * rpt + r * zrows_n,
                                               zrows_n)])
        plsc.subcore_barrier()
        for t in range(_QUAD):
            gather(0, t)

        def pair(p, _):
            qa = 2 * p
            qb = qa + 1
            gdesc_b = [gather_hi(qb, t) for t in range(_QUAD)]
            sdesc_a = []
            for t in range(_QUAD):
                pltpu.make_async_copy(ytab.at[sbuf.at[qa * _QUAD + t]],
                                      rows.at[t], gsem_a).wait()
                sdesc_a.append(scatter(qa, t, hi=False))
            for t in range(_QUAD):
                sdesc_a[t].wait()

            @pl.when(p + 1 < npair)
            def _():
                for t in range(_QUAD):
                    gather(2 * p + 2, t)
            sdesc_b = []
            for t in range(_QUAD):
                gdesc_b[t].wait()
                sdesc_b.append(scatter(qb, t, hi=True))
            for t in range(_QUAD):
                sdesc_b[t].wait()
            return 0
        lax.fori_loop(0, npair, pair, 0)
        plsc.subcore_barrier()
        pltpu.sync_copy(acc.at[pl.ds(s * rpt, rpt)],
                        out.at[c, pl.ds(s * rpt, rpt)])

    return pl.kernel(
        body,
        out_type=jax.ShapeDtypeStruct((_NC, acc_rows, hh), jnp.float32),
        mesh=_sc_mesh(),
        scratch_types=[
            pltpu.VMEM((cpw, _CHUNK), jnp.int32),
            pltpu.VMEM((cpw, _CHUNK), jnp.int32),
            pltpu.VMEM((2 * _QUAD, _CHUNK, hh), jnp.float32),
            pltpu.VMEM((rpt // 4, hh), jnp.float32),
            pltpu.VMEM_SHARED((n_nodes, hh), jnp.float32),
            pltpu.VMEM_SHARED((acc_rows, hh), jnp.float32),
            pltpu.SemaphoreType.DMA,
            pltpu.SemaphoreType.DMA,
            pltpu.SemaphoreType.DMA,
        ],
        compiler_params=pltpu.CompilerParams(use_tc_tiling_on_sc=False),
    )


def _xh_body(x_ref, w1_ref, b1_ref, w2_ref, b2_ref, o_ref):
    h = jnp.dot(x_ref[...], w1_ref[...], preferred_element_type=jnp.float32)
    h = jnp.maximum(h + b1_ref[...], 0.0)
    o_ref[...] = jnp.dot(h, w2_ref[...],
                         preferred_element_type=jnp.float32) + b2_ref[...]


def _scale_body(degp_ref, xh_ref, dis_ref, y3_ref):
    deg = degp_ref[0, :, 0:1] + degp_ref[1, :, 0:1]
    dis = jnp.where(deg > 0, 1.0 / jnp.sqrt(jnp.maximum(deg, 1e-12)), 0.0)
    dis_ref[...] = dis
    y = xh_ref[...] * dis
    hh = y.shape[1] // 2
    y3_ref[0] = y[:, :hh]
    y3_ref[1] = y[:, hh:]


def _combine_body(prev_ref, dis_ref, s_ref, b_ref, y3_ref):
    g = dis_ref[...] * jnp.concatenate([s_ref[0], s_ref[1]], axis=1)
    b = 0.5 * prev_ref[...] - 0.5 * g
    b_ref[...] = b
    y = dis_ref[...] * b
    hh = y.shape[1] // 2
    y3_ref[0] = y[:, :hh]
    y3_ref[1] = y[:, hh:]


def _fusion_body(xh_ref, b1_ref, dis_ref, s_ref,
                 wf_ref, bf_ref, wx_ref, bx_ref,
                 bw_ref, lam_ref, res_ref):
    xh = xh_ref[...]
    b1v = b1_ref[...]
    b2v = 0.5 * b1v - 0.5 * dis_ref[...] * jnp.concatenate(
        [s_ref[0], s_ref[1]], axis=1)
    basis = (xh, b1v, b2v)
    wv = jax.nn.sigmoid(bw_ref[...])
    xp = jnp.tanh(jnp.dot(xh, wx_ref[...],
                          preferred_element_type=jnp.float32) + bx_ref[...])
    hs, ls = [], []
    for f in range(5):
        h = jnp.zeros_like(xh)
        for i in range(3):
            coef = jnp.zeros((1, 1), jnp.float32)
            for k in range(3):
                ck = _BERN_COEFFS[k][i]
                if ck != 0.0:
                    coef = coef + ck * wv[f:f + 1, k:k + 1]
            h = h + coef * basis[i]
        hs.append(h)
        hp = jnp.tanh(jnp.dot(h, wf_ref[...],
                              preferred_element_type=jnp.float32) + bf_ref[...])
        ls.append(jnp.sum(hp * xp, axis=1, keepdims=True))
    logits = jnp.concatenate(ls, axis=1)
    m = jnp.max(logits, axis=1, keepdims=True)
    e = jnp.exp(logits - m)
    score = e / jnp.sum(e, axis=1, keepdims=True)
    sidx = lax.broadcasted_iota(jnp.int32, (1, 5), 1)
    sf = jnp.where(sidx == 0, 1.0, jax.nn.sigmoid(lam_ref[...]))
    res = jnp.zeros_like(xh)
    for f in range(5):
        res = res + sf[:, f:f + 1] * score[:, f:f + 1] * hs[f]
    res_ref[...] = res


def _gram_body(a_ref, b_ref, o_ref):
    prod = lax.dot_general(a_ref[...], b_ref[...],
                           (((1,), (1,)), ((), ())),
                           preferred_element_type=jnp.float32)
    o_ref[...] = jnp.tanh(prod)


def _full_spec(shape):
    return pl.BlockSpec(shape, lambda *_: tuple(0 for _ in shape))


def _row_spec(blk, shape):
    nd = len(shape)
    return pl.BlockSpec((blk,) + shape[1:],
                        lambda i: (i,) + tuple(0 for _ in range(nd - 1)))


def kernel(x, edge_index, W1, b1, W2, b2, bern_w, Wf, bf, Wx, bx, lam):
    n, in_ch = x.shape
    hid = W2.shape[1]
    e = edge_index.shape[1]

    epw = ((e + _NW * _CHUNK - 1) // (_NW * _CHUNK)) * _CHUNK
    e_pad = _NW * epw
    cpw = epw // _CHUNK
    src = edge_index[0]
    dst = edge_index[1]
    if e_pad > e:
        zpad = jnp.zeros((e_pad - e,), jnp.int32)
        src = jnp.concatenate([src, zpad])
        dst = jnp.concatenate([dst, zpad])
    src3 = src.reshape(_NW, cpw, _CHUNK)
    dst3 = dst.reshape(_NW, cpw, _CHUNK)
    cpw2 = e_pad // (_NS * _CHUNK)
    src3p = src.reshape(_NS, cpw2, _CHUNK)
    dst3p = dst.reshape(_NS, cpw2, _CHUNK)
    hh = hid // 2

    acc_rows = ((n + 1 + 127) // 128) * 128
    zeros16 = jnp.zeros((acc_rows, 16), jnp.float32)

    degp = _deg_kernel(n, cpw, acc_rows)(src3, dst3, zeros16)

    xh = pl.pallas_call(
        _xh_body,
        out_shape=jax.ShapeDtypeStruct((n, hid), jnp.float32),
        grid=(n // 2000,),
        in_specs=[_row_spec(2000, (n, in_ch)), _full_spec((in_ch, hid)),
                  _full_spec((1, hid)), _full_spec((hid, hid)),
                  _full_spec((1, hid))],
        out_specs=_row_spec(2000, (n, hid)),
    )(x, W1, b1.reshape(1, hid), W2, b2.reshape(1, hid))

    pair_spec = pl.BlockSpec((2, 2000, hh), lambda i: (0, i, 0))
    degp_spec = pl.BlockSpec((2, 2000, 16), lambda i: (0, i, 0))
    dis, y0 = pl.pallas_call(
        _scale_body,
        out_shape=[jax.ShapeDtypeStruct((n, 1), jnp.float32),
                   jax.ShapeDtypeStruct((2, n, hh), jnp.float32)],
        grid=(n // 2000,),
        in_specs=[degp_spec, _row_spec(2000, (n, hid))],
        out_specs=[_row_spec(2000, (n, 1)), pair_spec],
    )(degp, xh)

    s1 = _prop_kernel(n, cpw2, acc_rows, hid)(src3p, dst3p, y0)
    b1_arr, y1 = pl.pallas_call(
        _combine_body,
        out_shape=[jax.ShapeDtypeStruct((n, hid), jnp.float32),
                   jax.ShapeDtypeStruct((2, n, hh), jnp.float32)],
        grid=(n // 2000,),
        in_specs=[_row_spec(2000, (n, hid)), _row_spec(2000, (n, 1)),
                  pair_spec],
        out_specs=[_row_spec(2000, (n, hid)), pair_spec],
    )(xh, dis, s1)

    s2 = _prop_kernel(n, cpw2, acc_rows, hid)(src3p, dst3p, y1)
    res = pl.pallas_call(
        _fusion_body,
        out_shape=jax.ShapeDtypeStruct((n, hid), jnp.float32),
        grid=(n // 2000,),
        in_specs=[_row_spec(2000, (n, hid)), _row_spec(2000, (n, hid)),
                  _row_spec(2000, (n, 1)), pair_spec,
                  _full_spec((hid, hid)),
                  _full_spec((1, hid)), _full_spec((hid, hid)),
                  _full_spec((1, hid)), _full_spec((5, 3)),
                  _full_spec((1, 5))],
        out_specs=_row_spec(2000, (n, hid)),
    )(xh, b1_arr, dis, s2, Wf, bf.reshape(1, hid),
      Wx, bx.reshape(1, hid), bern_w.reshape(5, 3), lam.reshape(1, 5))

    bm = 400
    out = pl.pallas_call(
        _gram_body,
        out_shape=jax.ShapeDtypeStruct((n, n), jnp.float32),
        grid=(n // bm,),
        in_specs=[pl.BlockSpec((bm, hid), lambda i: (i, 0)),
                  pl.BlockSpec((n, hid), lambda i: (0, 0))],
        out_specs=pl.BlockSpec((bm, n), lambda i: (i, 0)),
    )(res, res)
    return out
